# Initial kernel scaffold; baseline (speedup 1.0000x reference)
import jax
import jax.numpy as jnp
from jax.experimental import pallas as pl


def kernel(x, edge_index, agent_mask, W_gcn, b_gcn, W1, b1, g1, be1, W2, b2, g2, be2, Wmu, bmu):
    raise NotImplementedError("write your pallas kernel here")


# speedup vs baseline: 15.4223x; 15.4223x over previous
import functools

import jax
import jax.numpy as jnp
from jax import lax
from jax.experimental import pallas as pl
from jax.experimental.pallas import tpu as pltpu
from jax.experimental.pallas import tpu_sc as plsc

N = 10000
E = 320000
D = 128
NAG = 1024
EPS = 1e-5

NC = 2
NS = 16
NW = NC * NS
EPT = E // NW
NPAD = 10240
NROW = 10016
CH = 2000
SLOTS_PER_TILE = NAG // NS

_mesh = plsc.VectorSubcoreMesh(core_axis_name="c", subcore_axis_name="s")
_sc_params = pltpu.CompilerParams(needs_layout_passes=False)


@functools.partial(
    pl.kernel,
    out_type=jax.ShapeDtypeStruct((NW, NPAD), jnp.float32),
    mesh=_mesh,
    compiler_params=_sc_params,
    scratch_types=[
        pltpu.VMEM((EPT,), jnp.int32),
        pltpu.VMEM((NPAD,), jnp.float32),
    ],
)
def _sc_degree(dst_hbm, out_hbm, dst_st, deg_v):
    c = lax.axis_index("c")
    s = lax.axis_index("s")
    wid = s * NC + c
    base = wid * EPT
    pltpu.sync_copy(dst_hbm.at[pl.ds(base, EPT)], dst_st)

    zeros = jnp.zeros((16,), jnp.float32)

    def _zero(i, carry):
        deg_v[pl.ds(i * 16, 16)] = zeros
        return carry

    lax.fori_loop(0, NPAD // 16, _zero, 0, unroll=8)

    ones = jnp.ones((16,), jnp.float32)

    def _count(i, carry):
        d = dst_st[pl.ds(i * 16, 16)]
        plsc.addupdate_scatter(deg_v, [d], ones)
        return carry

    lax.fori_loop(0, EPT // 16, _count, 0, unroll=4)
    pltpu.sync_copy(deg_v, out_hbm.at[wid])


def _tc_dinv_body(deg_ref, dinv_ref):
    s = jnp.sum(deg_ref[...], axis=0)
    dinv_ref[...] = lax.rsqrt(s + 1.0)


def _tc_dinv(deg_parts):
    return pl.pallas_call(
        _tc_dinv_body,
        out_shape=jax.ShapeDtypeStruct((NPAD // 128, 128), jnp.float32),
    )(deg_parts.reshape(NW, NPAD // 128, 128))


def _tc_scale_body(x_ref, dcol_ref, y_ref):
    y_ref[...] = x_ref[...] * dcol_ref[...]


def _tc_scale(x, dinv_col):
    return pl.pallas_call(
        _tc_scale_body,
        out_shape=jax.ShapeDtypeStruct((N, D), jnp.float32),
    )(x, dinv_col)


@functools.partial(
    pl.kernel,
    out_type=(
        jax.ShapeDtypeStruct((NC, NAG, D), jnp.float32),
        jax.ShapeDtypeStruct((NAG, D), jnp.float32),
        jax.ShapeDtypeStruct((NAG,), jnp.float32),
    ),
    mesh=_mesh,
    compiler_params=_sc_params,
    scratch_types=[
        pltpu.VMEM_SHARED((NROW, D), jnp.float32),
        pltpu.VMEM((CH,), jnp.int32),
        pltpu.VMEM((CH,), jnp.int32),
        pltpu.VMEM((16, D), jnp.float32),
        pltpu.VMEM((64, D), jnp.float32),
        pltpu.VMEM((NAG,), jnp.int32),
        pltpu.VMEM((NPAD,), jnp.float32),
        pltpu.VMEM((SLOTS_PER_TILE,), jnp.float32),
        pltpu.SemaphoreType.DMA,
    ],
)
def _sc_aggregate(src_hbm, dst_hbm, y_hbm, am_hbm, dinv_hbm,
                  acc_out, yrow_out, dinv_out,
                  acc_sh, src_st, dst_st, rows_v, zero_v, am_st, dinv_st,
                  dv_st, sem):
    c = lax.axis_index("c")
    s = lax.axis_index("s")
    wid = s * NC + c
    base = wid * EPT

    zeros = jnp.zeros((16,), jnp.float32)

    def _zero2(i, carry):
        r = i // 8
        k = i % 8
        zero_v[r, pl.ds(k * 16, 16)] = zeros
        return carry

    lax.fori_loop(0, 64 * D // 16, _zero2, 0, unroll=8)

    row0 = s * 626
    for j in range(9):
        pltpu.sync_copy(zero_v.at[pl.ds(0, 64)],
                        acc_sh.at[pl.ds(row0 + j * 64, 64)])
    pltpu.sync_copy(zero_v.at[pl.ds(0, 50)],
                    acc_sh.at[pl.ds(row0 + 576, 50)])
    plsc.subcore_barrier()

    pltpu.sync_copy(dinv_hbm, dinv_st)
    pltpu.sync_copy(am_hbm, am_st)

    for ch in range(EPT // CH):
        eb = base + ch * CH
        pltpu.sync_copy(src_hbm.at[pl.ds(eb, CH)], src_st)
        pltpu.sync_copy(dst_hbm.at[pl.ds(eb, CH)], dst_st)

        def _blk(i, carry):
            s16 = src_st[pl.ds(i * 16, 16)]
            d16 = dst_st[pl.ds(i * 16, 16)]
            pltpu.async_copy(y_hbm.at[s16], rows_v, sem).wait()
            pltpu.sync_copy(rows_v, acc_sh.at[d16], add=True)
            return carry

        lax.fori_loop(0, CH // 16, _blk, 0)

    plsc.subcore_barrier()

    slot0 = s * SLOTS_PER_TILE
    for g in range(SLOTS_PER_TILE // 16):
        ids = am_st[pl.ds(slot0 + g * 16, 16)]
        pltpu.sync_copy(acc_sh.at[ids], rows_v)
        pltpu.sync_copy(rows_v, acc_out.at[c, pl.ds(slot0 + g * 16, 16)])

    @pl.when(c == 0)
    def _core0_extras():
        for g in range(SLOTS_PER_TILE // 16):
            ids = am_st[pl.ds(slot0 + g * 16, 16)]
            pltpu.async_copy(y_hbm.at[ids], rows_v, sem).wait()
            pltpu.sync_copy(rows_v, yrow_out.at[pl.ds(slot0 + g * 16, 16)])
            dv_st[pl.ds(g * 16, 16)] = plsc.load_gather(dinv_st, [ids])
        pltpu.sync_copy(dv_st, dinv_out.at[pl.ds(slot0, SLOTS_PER_TILE)])


def _tc_head_body(acc_ref, yr_ref, dv_ref, wg_ref, bg_ref, w1_ref, b1_ref,
                  g1_ref, e1_ref, w2_ref, b2_ref, g2_ref, e2_ref, wm_ref,
                  bm_ref, out_ref):
    dv = dv_ref[...]
    rows = (acc_ref[0] + acc_ref[1] + yr_ref[...]) * dv
    h = jnp.dot(rows, wg_ref[...], preferred_element_type=jnp.float32)
    h = jnp.maximum(h + bg_ref[...], 0.0)
    z = jnp.dot(h, w1_ref[...], preferred_element_type=jnp.float32)
    z = z + b1_ref[...]
    m = jnp.mean(z, axis=-1, keepdims=True)
    v = jnp.mean((z - m) ** 2, axis=-1, keepdims=True)
    z = (z - m) * lax.rsqrt(v + EPS) * g1_ref[...] + e1_ref[...]
    z = jnp.maximum(z, 0.0)
    z2 = jnp.dot(z, w2_ref[...], preferred_element_type=jnp.float32)
    z2 = z2 + b2_ref[...]
    m = jnp.mean(z2, axis=-1, keepdims=True)
    v = jnp.mean((z2 - m) ** 2, axis=-1, keepdims=True)
    z2 = (z2 - m) * lax.rsqrt(v + EPS) * g2_ref[...] + e2_ref[...]
    z2 = jnp.maximum(z2, 0.0)
    o = jnp.dot(z2, wm_ref[...], preferred_element_type=jnp.float32)
    out_ref[...] = jax.nn.sigmoid(o + bm_ref[...])


def _tc_head(acc, yrows, dinv_col, W_gcn, b_gcn, W1, b1, g1, be1,
             W2, b2, g2, be2, Wmu, bmu):
    return pl.pallas_call(
        _tc_head_body,
        out_shape=jax.ShapeDtypeStruct((NAG, Wmu.shape[1]), jnp.float32),
    )(acc, yrows, dinv_col,
      W_gcn, b_gcn.reshape(1, -1), W1, b1.reshape(1, -1),
      g1.reshape(1, -1), be1.reshape(1, -1), W2, b2.reshape(1, -1),
      g2.reshape(1, -1), be2.reshape(1, -1), Wmu, bmu.reshape(1, -1))


def kernel(x, edge_index, agent_mask, W_gcn, b_gcn, W1, b1, g1, be1,
           W2, b2, g2, be2, Wmu, bmu):
    src = edge_index[0]
    dst = edge_index[1]
    deg_parts = _sc_degree(dst)
    dinv2d = _tc_dinv(deg_parts)
    dinv_flat = dinv2d.reshape(NPAD)
    y = _tc_scale(x, dinv_flat[:N].reshape(N, 1))
    acc, yrows, dinv_ag = _sc_aggregate(src, dst, y, agent_mask, dinv_flat)
    return _tc_head(acc, yrows, dinv_ag.reshape(NAG, 1),
                    W_gcn, b_gcn, W1, b1, g1, be1, W2, b2, g2, be2, Wmu, bmu)


# speedup vs baseline: 63.9809x; 4.1486x over previous
import functools

import jax
import jax.numpy as jnp
from jax import lax
from jax.experimental import pallas as pl
from jax.experimental.pallas import tpu as pltpu
from jax.experimental.pallas import tpu_sc as plsc

N = 10000
E = 320000
D = 128
NAG = 1024
EPS = 1e-5

NC = 2
NS = 16
NW = NC * NS
EPT = E // NW
NPAD = 10240
NROW = 10016
CH = 2000
SLOTS_PER_TILE = NAG // NS

_mesh = plsc.VectorSubcoreMesh(core_axis_name="c", subcore_axis_name="s")
_sc_params = pltpu.CompilerParams(needs_layout_passes=False)


@functools.partial(
    pl.kernel,
    out_type=jax.ShapeDtypeStruct((NW, NPAD), jnp.float32),
    mesh=_mesh,
    compiler_params=_sc_params,
    scratch_types=[
        pltpu.VMEM((EPT,), jnp.int32),
        pltpu.VMEM((NPAD,), jnp.float32),
    ],
)
def _sc_degree(dst_hbm, out_hbm, dst_st, deg_v):
    c = lax.axis_index("c")
    s = lax.axis_index("s")
    wid = s * NC + c
    base = wid * EPT
    pltpu.sync_copy(dst_hbm.at[pl.ds(base, EPT)], dst_st)

    zeros = jnp.zeros((16,), jnp.float32)

    def _zero(i, carry):
        deg_v[pl.ds(i * 16, 16)] = zeros
        return carry

    lax.fori_loop(0, NPAD // 16, _zero, 0, unroll=8)

    ones = jnp.ones((16,), jnp.float32)

    def _count(i, carry):
        d = dst_st[pl.ds(i * 16, 16)]
        plsc.addupdate_scatter(deg_v, [d], ones)
        return carry

    lax.fori_loop(0, EPT // 16, _count, 0, unroll=4)
    pltpu.sync_copy(deg_v, out_hbm.at[wid])


def _tc_dinv_body(deg_ref, dinv_ref):
    s = jnp.sum(deg_ref[...], axis=0)
    dinv_ref[...] = lax.rsqrt(s + 1.0)


def _tc_dinv(deg_parts):
    return pl.pallas_call(
        _tc_dinv_body,
        out_shape=jax.ShapeDtypeStruct((NPAD // 128, 128), jnp.float32),
    )(deg_parts.reshape(NW, NPAD // 128, 128))


def _tc_scale_body(x_ref, dcol_ref, y_ref):
    y_ref[...] = x_ref[...] * dcol_ref[...]


def _tc_scale(x, dinv_col):
    return pl.pallas_call(
        _tc_scale_body,
        out_shape=jax.ShapeDtypeStruct((N, D), jnp.float32),
    )(x, dinv_col)


NROW2 = 2048
DUMMY = NAG


@functools.partial(
    pl.kernel,
    out_type=(
        jax.ShapeDtypeStruct((NC, NAG, D), jnp.float32),
        jax.ShapeDtypeStruct((NAG, D), jnp.float32),
        jax.ShapeDtypeStruct((NAG,), jnp.float32),
    ),
    mesh=_mesh,
    compiler_params=_sc_params,
    scratch_types=[
        pltpu.VMEM_SHARED((NROW2, D), jnp.float32),
        pltpu.VMEM((EPT,), jnp.int32),
        pltpu.VMEM((EPT,), jnp.int32),
        pltpu.VMEM((EPT + 16,), jnp.int32),
        pltpu.VMEM((EPT + 16,), jnp.int32),
        pltpu.VMEM((NPAD,), jnp.int32),
        pltpu.VMEM((16, D), jnp.float32),
        pltpu.VMEM((64, D), jnp.float32),
        pltpu.VMEM((NAG,), jnp.int32),
        pltpu.VMEM((NPAD,), jnp.float32),
        pltpu.VMEM((SLOTS_PER_TILE,), jnp.float32),
        pltpu.SemaphoreType.DMA,
    ],
)
def _sc_aggregate(src_hbm, dst_hbm, y_hbm, am_hbm, dinv_hbm,
                  acc_out, yrow_out, dinv_out,
                  acc_sh, src_st, dst_st, sel_src, sel_rnk, rank_st,
                  rows_v, zero_v, am_st, dinv_st, dv_st, sem):
    c = lax.axis_index("c")
    s = lax.axis_index("s")
    wid = s * NC + c
    base = wid * EPT

    zeros = jnp.zeros((16,), jnp.float32)

    def _zero2(i, carry):
        r = i // 8
        k = i % 8
        zero_v[r, pl.ds(k * 16, 16)] = zeros
        return carry

    lax.fori_loop(0, 64 * D // 16, _zero2, 0, unroll=8)

    for j in range(2):
        pltpu.sync_copy(zero_v, acc_sh.at[pl.ds(s * 128 + j * 64, 64)])
    plsc.subcore_barrier()

    pltpu.sync_copy(dinv_hbm, dinv_st)
    pltpu.sync_copy(am_hbm, am_st)
    pltpu.sync_copy(src_hbm.at[pl.ds(base, EPT)], src_st)
    pltpu.sync_copy(dst_hbm.at[pl.ds(base, EPT)], dst_st)

    dummies = jnp.full((16,), DUMMY, jnp.int32)

    def _zr(i, carry):
        rank_st[pl.ds(i * 16, 16)] = dummies
        return carry

    lax.fori_loop(0, NPAD // 16, _zr, 0, unroll=8)

    lane = lax.iota(jnp.int32, 16)

    def _sr(i, carry):
        ids = am_st[pl.ds(i * 16, 16)]
        plsc.store_scatter(rank_st, [ids], lane + i * 16)
        return carry

    lax.fori_loop(0, NAG // 16, _sr, 0, unroll=4)

    def _cmp(i, fill):
        s16 = src_st[pl.ds(i * 16, 16)]
        d16 = dst_st[pl.ds(i * 16, 16)]
        r16 = plsc.load_gather(rank_st, [d16])
        m = r16 < NAG
        plsc.store_compressed(sel_src.at[pl.ds(fill, 16)], s16, mask=m)
        plsc.store_compressed(sel_rnk.at[pl.ds(fill, 16)], r16, mask=m)
        return fill + plsc.all_reduce_population_count(m)[0]

    fill = lax.fori_loop(0, EPT // 16, _cmp, jnp.int32(0), unroll=2)
    sel_src[pl.ds(fill, 16)] = jnp.zeros((16,), jnp.int32)
    sel_rnk[pl.ds(fill, 16)] = dummies
    nblk = (fill + 15) // 16

    def _blk(i, carry):
        s16 = sel_src[pl.ds(i * 16, 16)]
        r16 = sel_rnk[pl.ds(i * 16, 16)]
        pltpu.async_copy(y_hbm.at[s16], rows_v, sem).wait()
        pltpu.sync_copy(rows_v, acc_sh.at[r16], add=True)
        return carry

    lax.fori_loop(0, nblk, _blk, 0)

    plsc.subcore_barrier()

    slot0 = s * SLOTS_PER_TILE
    for g in range(SLOTS_PER_TILE // 16):
        ids = am_st[pl.ds(slot0 + g * 16, 16)]
        r16 = plsc.load_gather(rank_st, [ids])
        pltpu.sync_copy(acc_sh.at[r16], rows_v)
        pltpu.sync_copy(rows_v, acc_out.at[c, pl.ds(slot0 + g * 16, 16)])

    @pl.when(c == 0)
    def _core0_extras():
        for g in range(SLOTS_PER_TILE // 16):
            ids = am_st[pl.ds(slot0 + g * 16, 16)]
            pltpu.async_copy(y_hbm.at[ids], rows_v, sem).wait()
            pltpu.sync_copy(rows_v, yrow_out.at[pl.ds(slot0 + g * 16, 16)])
            dv_st[pl.ds(g * 16, 16)] = plsc.load_gather(dinv_st, [ids])
        pltpu.sync_copy(dv_st, dinv_out.at[pl.ds(slot0, SLOTS_PER_TILE)])


def _tc_head_body(acc_ref, yr_ref, dv_ref, wg_ref, bg_ref, w1_ref, b1_ref,
                  g1_ref, e1_ref, w2_ref, b2_ref, g2_ref, e2_ref, wm_ref,
                  bm_ref, out_ref):
    dv = dv_ref[...]
    rows = (acc_ref[0] + acc_ref[1] + yr_ref[...]) * dv
    h = jnp.dot(rows, wg_ref[...], preferred_element_type=jnp.float32)
    h = jnp.maximum(h + bg_ref[...], 0.0)
    z = jnp.dot(h, w1_ref[...], preferred_element_type=jnp.float32)
    z = z + b1_ref[...]
    m = jnp.mean(z, axis=-1, keepdims=True)
    v = jnp.mean((z - m) ** 2, axis=-1, keepdims=True)
    z = (z - m) * lax.rsqrt(v + EPS) * g1_ref[...] + e1_ref[...]
    z = jnp.maximum(z, 0.0)
    z2 = jnp.dot(z, w2_ref[...], preferred_element_type=jnp.float32)
    z2 = z2 + b2_ref[...]
    m = jnp.mean(z2, axis=-1, keepdims=True)
    v = jnp.mean((z2 - m) ** 2, axis=-1, keepdims=True)
    z2 = (z2 - m) * lax.rsqrt(v + EPS) * g2_ref[...] + e2_ref[...]
    z2 = jnp.maximum(z2, 0.0)
    o = jnp.dot(z2, wm_ref[...], preferred_element_type=jnp.float32)
    out_ref[...] = jax.nn.sigmoid(o + bm_ref[...])


def _tc_head(acc, yrows, dinv_col, W_gcn, b_gcn, W1, b1, g1, be1,
             W2, b2, g2, be2, Wmu, bmu):
    return pl.pallas_call(
        _tc_head_body,
        out_shape=jax.ShapeDtypeStruct((NAG, Wmu.shape[1]), jnp.float32),
    )(acc, yrows, dinv_col,
      W_gcn, b_gcn.reshape(1, -1), W1, b1.reshape(1, -1),
      g1.reshape(1, -1), be1.reshape(1, -1), W2, b2.reshape(1, -1),
      g2.reshape(1, -1), be2.reshape(1, -1), Wmu, bmu.reshape(1, -1))


def kernel(x, edge_index, agent_mask, W_gcn, b_gcn, W1, b1, g1, be1,
           W2, b2, g2, be2, Wmu, bmu):
    src = edge_index[0]
    dst = edge_index[1]
    deg_parts = _sc_degree(dst)
    dinv2d = _tc_dinv(deg_parts)
    dinv_flat = dinv2d.reshape(NPAD)
    y = _tc_scale(x, dinv_flat[:N].reshape(N, 1))
    acc, yrows, dinv_ag = _sc_aggregate(src, dst, y, agent_mask, dinv_flat)
    return _tc_head(acc, yrows, dinv_ag.reshape(NAG, 1),
                    W_gcn, b_gcn, W1, b1, g1, be1, W2, b2, g2, be2, Wmu, bmu)
